# Initial kernel scaffold; baseline (speedup 1.0000x reference)
import jax
import jax.numpy as jnp
from jax.experimental import pallas as pl


def kernel(x, batch, edge_index, W1_0, b1_0, W2_0, b2_0, W1_1, b1_1, W2_1, b2_1):
    raise NotImplementedError("write your pallas kernel here")


# speedup vs baseline: 5.1083x; 5.1083x over previous
import functools

import jax
import jax.numpy as jnp
from jax import lax
from jax.experimental import pallas as pl
from jax.experimental.pallas import tpu as pltpu
from jax.experimental.pallas import tpu_sc as plsc

N = 10000
E = 320000
D = 128

NC = 2
NS = 16
NW = NC * NS
EDGES_PER_TILE = E // NW
CHUNK = 80
NCHUNK = EDGES_PER_TILE // CHUNK
ROWS_PER_TILE = 624
TAIL_ROWS = N - NS * ROWS_PER_TILE


def _sc_segment_sum(h, src, dst, zeros):
    mesh = plsc.VectorSubcoreMesh(core_axis_name="c", subcore_axis_name="s")

    @functools.partial(
        pl.kernel,
        out_type=jax.ShapeDtypeStruct((NC, N, D), jnp.float32),
        mesh=mesh,
        scratch_types=[
            pltpu.VMEM_SHARED((N, D), jnp.float32),
            pltpu.VMEM((CHUNK,), jnp.int32),
            pltpu.VMEM((CHUNK,), jnp.int32),
            pltpu.VMEM((CHUNK, D), jnp.float32),
            pltpu.SemaphoreType.DMA,
        ],
    )
    def k(h_hbm, src_hbm, dst_hbm, z_hbm, out_hbm, acc, sidx, didx, rows, sem):
        c = lax.axis_index("c")
        s = lax.axis_index("s")
        wid = c * NS + s

        pltpu.sync_copy(z_hbm.at[pl.ds(s * ROWS_PER_TILE, ROWS_PER_TILE)],
                        acc.at[pl.ds(s * ROWS_PER_TILE, ROWS_PER_TILE)])

        @pl.when(s == NS - 1)
        def _zero_tail():
            pltpu.sync_copy(z_hbm.at[pl.ds(NS * ROWS_PER_TILE, TAIL_ROWS)],
                            acc.at[pl.ds(NS * ROWS_PER_TILE, TAIL_ROWS)])

        plsc.subcore_barrier()

        def chunk_body(i, carry):
            base = wid * EDGES_PER_TILE + i * CHUNK
            pltpu.sync_copy(src_hbm.at[pl.ds(base, CHUNK)], sidx)
            pltpu.sync_copy(dst_hbm.at[pl.ds(base, CHUNK)], didx)
            pltpu.async_copy(h_hbm.at[sidx], rows, sem).wait()
            pltpu.sync_copy(rows, acc.at[didx], add=True)
            return carry

        lax.fori_loop(0, NCHUNK, chunk_body, 0)
        plsc.subcore_barrier()

        pltpu.sync_copy(acc.at[pl.ds(s * ROWS_PER_TILE, ROWS_PER_TILE)],
                        out_hbm.at[c, pl.ds(s * ROWS_PER_TILE, ROWS_PER_TILE)])

        @pl.when(s == NS - 1)
        def _copy_tail():
            pltpu.sync_copy(acc.at[pl.ds(NS * ROWS_PER_TILE, TAIL_ROWS)],
                            out_hbm.at[c, pl.ds(NS * ROWS_PER_TILE, TAIL_ROWS)])

    return k(h, src, dst, zeros)


BN = 1000


def _ffn_body(h_ref, p_ref, w1_ref, b1_ref, w2_ref, b2_ref, o_ref):
    h = h_ref[...] + p_ref[0] + p_ref[1]
    t = jnp.dot(h, w1_ref[...], preferred_element_type=jnp.float32) + b1_ref[...]
    t = jnp.maximum(t, 0.01 * t)
    o_ref[...] = jnp.dot(t, w2_ref[...], preferred_element_type=jnp.float32) + b2_ref[...]


def _ffn_skip_body(h_ref, p_ref, w1_ref, b1_ref, w2_ref, b2_ref, x0_ref, o_ref):
    h = h_ref[...] + p_ref[0] + p_ref[1]
    t = jnp.dot(h, w1_ref[...], preferred_element_type=jnp.float32) + b1_ref[...]
    t = jnp.maximum(t, 0.01 * t)
    o_ref[...] = (jnp.dot(t, w2_ref[...], preferred_element_type=jnp.float32)
                  + b2_ref[...] + x0_ref[...])


_ROW_SPEC = pl.BlockSpec((BN, D), lambda i: (i, 0))
_PART_SPEC = pl.BlockSpec((NC, BN, D), lambda i: (0, i, 0))
_W_SPEC = pl.BlockSpec((D, D), lambda i: (0, 0))
_B_SPEC = pl.BlockSpec((1, D), lambda i: (0, 0))


def _tc_ffn(h, parts, w1, b1, w2, b2):
    return pl.pallas_call(
        _ffn_body,
        grid=(N // BN,),
        in_specs=[_ROW_SPEC, _PART_SPEC, _W_SPEC, _B_SPEC, _W_SPEC, _B_SPEC],
        out_specs=_ROW_SPEC,
        out_shape=jax.ShapeDtypeStruct((N, D), jnp.float32),
    )(h, parts, w1, b1.reshape(1, D), w2, b2.reshape(1, D))


def _tc_ffn_skip(h, parts, w1, b1, w2, b2, x0):
    return pl.pallas_call(
        _ffn_skip_body,
        grid=(N // BN,),
        in_specs=[_ROW_SPEC, _PART_SPEC, _W_SPEC, _B_SPEC, _W_SPEC, _B_SPEC,
                  _ROW_SPEC],
        out_specs=_ROW_SPEC,
        out_shape=jax.ShapeDtypeStruct((N, D), jnp.float32),
    )(h, parts, w1, b1.reshape(1, D), w2, b2.reshape(1, D), x0)


def kernel(x, batch, edge_index, W1_0, b1_0, W2_0, b2_0, W1_1, b1_1, W2_1, b2_1):
    src = edge_index[0]
    dst = edge_index[1]
    zeros = jnp.zeros((N, D), jnp.float32)

    parts1 = _sc_segment_sum(x, src, dst, zeros)
    h1 = _tc_ffn(x, parts1, W1_0, b1_0, W2_0, b2_0)
    parts2 = _sc_segment_sum(h1, src, dst, zeros)
    return _tc_ffn_skip(h1, parts2, W1_1, b1_1, W2_1, b2_1, x)


# speedup vs baseline: 10.2915x; 2.0147x over previous
import functools

import jax
import jax.numpy as jnp
from jax import lax
from jax.experimental import pallas as pl
from jax.experimental.pallas import tpu as pltpu
from jax.experimental.pallas import tpu_sc as plsc

N = 10000
E = 320000
D = 128

NC = 2
NS = 16
NW = NC * NS
CHUNK = 128
NCH_ALL = E // CHUNK
CH_PER_TILE = 80
NBUF = 2
ROWS_PER_TILE = 624
TAIL_ROWS = N - NS * ROWS_PER_TILE


def _sc_segment_sum(h, src, dst2d, zeros):
    mesh = plsc.VectorSubcoreMesh(core_axis_name="c", subcore_axis_name="s")

    @functools.partial(
        pl.kernel,
        out_type=jax.ShapeDtypeStruct((NC, N, D), jnp.float32),
        mesh=mesh,
        scratch_types=[
            pltpu.VMEM_SHARED((N, D), jnp.float32),
            pltpu.VMEM((NBUF, CHUNK), jnp.int32),
            pltpu.VMEM((CH_PER_TILE, CHUNK), jnp.int32),
            pltpu.VMEM((NBUF, CHUNK, D), jnp.float32),
        ] + [pltpu.SemaphoreType.DMA] * (2 * NBUF),
    )
    def k(h_hbm, src_hbm, dst2d_hbm, z_hbm, out_hbm, acc, sidx, didx, rows,
          *sems):
        gsem = sems[:NBUF]
        ssem = sems[NBUF:]
        c = lax.axis_index("c")
        s = lax.axis_index("s")
        wid = c * NS + s
        is_last = wid == NW - 1
        ntail = NCH_ALL - (NW - 1) * CH_PER_TILE
        nch = jnp.where(is_last, ntail, CH_PER_TILE)

        pltpu.sync_copy(z_hbm.at[pl.ds(s * ROWS_PER_TILE, ROWS_PER_TILE)],
                        acc.at[pl.ds(s * ROWS_PER_TILE, ROWS_PER_TILE)])

        @pl.when(s == NS - 1)
        def _zero_tail():
            pltpu.sync_copy(z_hbm.at[pl.ds(NS * ROWS_PER_TILE, TAIL_ROWS)],
                            acc.at[pl.ds(NS * ROWS_PER_TILE, TAIL_ROWS)])

        @pl.when(jnp.logical_not(is_last))
        def _stage_idx_full():
            pltpu.sync_copy(dst2d_hbm.at[pl.ds(wid * CH_PER_TILE, CH_PER_TILE)],
                            didx)

        @pl.when(is_last)
        def _stage_idx_tail():
            pltpu.sync_copy(dst2d_hbm.at[pl.ds((NW - 1) * CH_PER_TILE, ntail)],
                            didx.at[pl.ds(0, ntail)])

        edge_base = wid * CH_PER_TILE * CHUNK

        def _sidx_copy(ci, b):
            pltpu.sync_copy(src_hbm.at[pl.ds(edge_base + ci * CHUNK, CHUNK)],
                            sidx.at[b])

        def _gstart(ci, b):
            pltpu.async_copy(h_hbm.at[sidx.at[b]], rows.at[b], gsem[b])

        def _gwait(ci, b):
            pltpu.make_async_copy(h_hbm.at[sidx.at[b]], rows.at[b],
                                  gsem[b]).wait()

        def _sstart(ci, b):
            pltpu.async_copy(rows.at[b], acc.at[didx.at[ci]], ssem[b],
                             add=True)

        def _swait(ci, b):
            pltpu.make_async_copy(rows.at[b], acc.at[didx.at[ci]],
                                  ssem[b]).wait()

        plsc.subcore_barrier()

        for b in range(NBUF):
            _sidx_copy(b, b)
            _gstart(b, b)

        def pipe_body(g, carry):
            for b in range(NBUF):
                ci = g * NBUF + b
                _gwait(ci, b)
                _sstart(ci, b)
            for b in range(NBUF):
                nci = g * NBUF + b + NBUF

                @pl.when(nci < nch)
                def _():
                    _swait(nci - NBUF, b)
                    _sidx_copy(nci, b)
                    _gstart(nci, b)

            return carry

        lax.fori_loop(0, nch // NBUF, pipe_body, 0)
        for b in range(NBUF):
            _swait(nch - NBUF + b, b)

        plsc.subcore_barrier()

        pltpu.sync_copy(acc.at[pl.ds(s * ROWS_PER_TILE, ROWS_PER_TILE)],
                        out_hbm.at[c, pl.ds(s * ROWS_PER_TILE, ROWS_PER_TILE)])

        @pl.when(s == NS - 1)
        def _copy_tail():
            pltpu.sync_copy(acc.at[pl.ds(NS * ROWS_PER_TILE, TAIL_ROWS)],
                            out_hbm.at[c, pl.ds(NS * ROWS_PER_TILE, TAIL_ROWS)])

    return k(h, src, dst2d, zeros)


BN = 1000


def _ffn_body(h_ref, p_ref, w1_ref, b1_ref, w2_ref, b2_ref, o_ref):
    h = h_ref[...] + p_ref[0] + p_ref[1]
    t = jnp.dot(h, w1_ref[...], preferred_element_type=jnp.float32) + b1_ref[...]
    t = jnp.maximum(t, 0.01 * t)
    o_ref[...] = jnp.dot(t, w2_ref[...], preferred_element_type=jnp.float32) + b2_ref[...]


def _ffn_skip_body(h_ref, p_ref, w1_ref, b1_ref, w2_ref, b2_ref, x0_ref, o_ref):
    h = h_ref[...] + p_ref[0] + p_ref[1]
    t = jnp.dot(h, w1_ref[...], preferred_element_type=jnp.float32) + b1_ref[...]
    t = jnp.maximum(t, 0.01 * t)
    o_ref[...] = (jnp.dot(t, w2_ref[...], preferred_element_type=jnp.float32)
                  + b2_ref[...] + x0_ref[...])


_ROW_SPEC = pl.BlockSpec((BN, D), lambda i: (i, 0))
_PART_SPEC = pl.BlockSpec((NC, BN, D), lambda i: (0, i, 0))
_W_SPEC = pl.BlockSpec((D, D), lambda i: (0, 0))
_B_SPEC = pl.BlockSpec((1, D), lambda i: (0, 0))


def _tc_ffn(h, parts, w1, b1, w2, b2):
    return pl.pallas_call(
        _ffn_body,
        grid=(N // BN,),
        in_specs=[_ROW_SPEC, _PART_SPEC, _W_SPEC, _B_SPEC, _W_SPEC, _B_SPEC],
        out_specs=_ROW_SPEC,
        out_shape=jax.ShapeDtypeStruct((N, D), jnp.float32),
    )(h, parts, w1, b1.reshape(1, D), w2, b2.reshape(1, D))


def _tc_ffn_skip(h, parts, w1, b1, w2, b2, x0):
    return pl.pallas_call(
        _ffn_skip_body,
        grid=(N // BN,),
        in_specs=[_ROW_SPEC, _PART_SPEC, _W_SPEC, _B_SPEC, _W_SPEC, _B_SPEC,
                  _ROW_SPEC],
        out_specs=_ROW_SPEC,
        out_shape=jax.ShapeDtypeStruct((N, D), jnp.float32),
    )(h, parts, w1, b1.reshape(1, D), w2, b2.reshape(1, D), x0)


def kernel(x, batch, edge_index, W1_0, b1_0, W2_0, b2_0, W1_1, b1_1, W2_1, b2_1):
    src = edge_index[0]
    dst2d = edge_index[1].reshape(NCH_ALL, CHUNK)
    zeros = jnp.zeros((N, D), jnp.float32)

    parts1 = _sc_segment_sum(x, src, dst2d, zeros)
    h1 = _tc_ffn(x, parts1, W1_0, b1_0, W2_0, b2_0)
    parts2 = _sc_segment_sum(h1, src, dst2d, zeros)
    return _tc_ffn_skip(h1, parts2, W1_1, b1_1, W2_1, b2_1, x)
